# Initial kernel scaffold; baseline (speedup 1.0000x reference)
import jax
import jax.numpy as jnp
from jax.experimental import pallas as pl


def kernel(x, emb, edge_index, pedge_index, Wn, bn, We, be, W_conv0, V_conv0, b_conv0, g_bn0, beta_bn0, W_conv1, V_conv1, b_conv1, g_bn1, beta_bn1, W1, b1, W2, b2):
    raise NotImplementedError("write your pallas kernel here")


# speedup vs baseline: 4.4840x; 4.4840x over previous
import dataclasses
import functools

import jax
import jax.numpy as jnp
from jax import lax
from jax.experimental import pallas as pl
from jax.experimental.pallas import tpu as pltpu
from jax.experimental.pallas import tpu_sc as plsc

N = 10000
E = 320000
HID = 50
HG = 128
HR = 64
NC = 2
NS = 16
NW = NC * NS
EPW = E // NW
CH = 80
NCHUNK = EPW // CH
NP = 10240
RPT = NP // NS
ZR = 128
SB = 160


def _mesh():
    return plsc.VectorSubcoreMesh(
        core_axis_name="c", subcore_axis_name="s", num_cores=NC,
        num_subcores=NS)


def _sc_params():
    cp = pltpu.CompilerParams()
    if "needs_layout_passes" in pltpu.CompilerParams.__dataclass_fields__:
        cp = dataclasses.replace(cp, needs_layout_passes=False)
    return cp


def _z16():
    return jnp.zeros((16,), jnp.float32)


def _sc_degree(dst1):
    @functools.partial(
        pl.kernel,
        out_type=jax.ShapeDtypeStruct((NW, 1, NP), jnp.float32),
        mesh=_mesh(),
        compiler_params=_sc_params(),
        scratch_types=[
            pltpu.VMEM((EPW,), jnp.int32),
            pltpu.VMEM((NP,), jnp.float32),
        ],
    )
    def deg_kernel(dst_hbm, out_hbm, idxf, deg_t):
        cid = lax.axis_index("c")
        sid = lax.axis_index("s")
        wid = cid * NS + sid

        @pl.loop(0, NP // 16)
        def _(i):
            deg_t[pl.ds(i * 16, 16)] = _z16()

        pltpu.sync_copy(dst_hbm.at[pl.ds(wid * EPW, EPW)], idxf)
        ones16 = jnp.full((16,), 1.0, jnp.float32)

        @pl.loop(0, EPW // 16)
        def _(k):
            iv = idxf[pl.ds(k * 16, 16)]
            plsc.addupdate_scatter(deg_t, [iv], ones16)

        pltpu.sync_copy(deg_t, out_hbm.at[wid, 0])

    return deg_kernel(dst1)


def _sc_segsum(g, src1, dst1):
    @functools.partial(
        pl.kernel,
        out_type=jax.ShapeDtypeStruct((NC, NS, RPT // 2, HG), jnp.float32),
        mesh=_mesh(),
        compiler_params=_sc_params(),
        scratch_types=[
            pltpu.VMEM((EPW,), jnp.int32),
            pltpu.VMEM((EPW,), jnp.int32),
            pltpu.VMEM((CH,), jnp.int32),
            pltpu.VMEM((CH, HG), jnp.float32),
            pltpu.VMEM((CH, HG), jnp.float32),
            pltpu.VMEM((ZR, HG), jnp.float32),
            pltpu.VMEM_SHARED((NP // 2, HG), jnp.float32),
            pltpu.SemaphoreType.DMA,
        ],
    )
    def seg_kernel(g_hbm, src_hbm, dst_hbm, out_hbm,
                   idxs, idxd, idx80, rows_v, rows_p, zero_v,
                   acc_sh, sem):
        cid = lax.axis_index("c")
        sid = lax.axis_index("s")
        wid = cid * NS + sid
        rpt2 = RPT // 2

        @pl.loop(0, ZR)
        def _(r):
            for c in range(HG // 16):
                zero_v[r, pl.ds(c * 16, 16)] = _z16()

        for k in range(rpt2 // ZR):
            pltpu.sync_copy(
                zero_v, acc_sh.at[pl.ds(sid * rpt2 + k * ZR, ZR)])
        pltpu.sync_copy(
            zero_v.at[pl.ds(0, rpt2 - (rpt2 // ZR) * ZR)],
            acc_sh.at[pl.ds(sid * rpt2 + (rpt2 // ZR) * ZR,
                            rpt2 - (rpt2 // ZR) * ZR)])
        plsc.subcore_barrier()

        pltpu.sync_copy(src_hbm.at[pl.ds(wid * EPW, EPW)], idxs)
        pltpu.sync_copy(dst_hbm.at[pl.ds(wid * EPW, EPW)], idxd)

        def chunk(j, _):
            off = pl.multiple_of(j * CH, 8)
            pltpu.async_copy(
                g_hbm.at[idxs.at[pl.ds(off, CH)]], rows_v, sem).wait()

            def packrow(r, _):
                v16 = idxd[pl.ds(jnp.minimum(j * CH + r, EPW - 16), 16)]
                pos = j * CH + r - jnp.minimum(j * CH + r, EPW - 16)
                dv = jnp.sum(jnp.where(
                    lax.iota(jnp.int32, 16) == pos, v16, 0))
                ho = (dv & 1) * HR
                oo = HR - ho
                for c in range(HR // 16):
                    rows_p[r, pl.ds(ho + c * 16, 16)] = \
                        rows_v[r, pl.ds(c * 16, 16)]
                    rows_p[r, pl.ds(oo + c * 16, 16)] = \
                        rows_v[r, pl.ds(HR + c * 16, 16)]
                return 0

            lax.fori_loop(0, CH, packrow, 0)

            for k in range(CH // 16):
                idx80[pl.ds(k * 16, 16)] = lax.shift_right_logical(
                    idxd[pl.ds(pl.multiple_of(j * CH + k * 16, 8), 16)], 1)

            pltpu.sync_copy(rows_p, acc_sh.at[idx80], add=True)
            return 0

        lax.fori_loop(0, NCHUNK, chunk, 0)
        plsc.subcore_barrier()

        for b in range(rpt2 // ZR):
            pltpu.sync_copy(acc_sh.at[pl.ds(sid * rpt2 + b * ZR, ZR)],
                            zero_v)
            pltpu.sync_copy(zero_v, out_hbm.at[cid, sid, pl.ds(b * ZR, ZR)])
        rem = rpt2 - (rpt2 // ZR) * ZR
        pltpu.sync_copy(
            acc_sh.at[pl.ds(sid * rpt2 + (rpt2 // ZR) * ZR, rem)],
            zero_v.at[pl.ds(0, rem)])
        pltpu.sync_copy(zero_v.at[pl.ds(0, rem)],
                        out_hbm.at[cid, sid, pl.ds((rpt2 // ZR) * ZR, rem)])

    return seg_kernel(g, src1, dst1)


def _sc_edge1(P, idx1):
    @functools.partial(
        pl.kernel,
        out_type=jax.ShapeDtypeStruct((NW, NCHUNK, CH // 2, HG),
                                      jnp.float32),
        mesh=_mesh(),
        compiler_params=_sc_params(),
        scratch_types=[
            pltpu.VMEM((EPW,), jnp.int32),
            pltpu.VMEM((CH, HG), jnp.float32),
            pltpu.VMEM((CH // 2, HG), jnp.float32),
        ],
    )
    def edge_kernel(p_hbm, src_hbm, out_hbm, idxs, bufp, bufr2):
        cid = lax.axis_index("c")
        sid = lax.axis_index("s")
        wid = cid * NS + sid

        pltpu.sync_copy(src_hbm.at[pl.ds(wid * EPW, EPW)], idxs)

        for j in range(NCHUNK):
            pltpu.sync_copy(p_hbm.at[idxs.at[pl.ds(j * CH, CH)]], bufp)

            def addrow(r2, _):
                for half in range(2):
                    for c in range(HR // 16):
                        bufr2[r2, pl.ds(half * HR + c * 16, 16)] = \
                            bufp[2 * r2 + half, pl.ds(c * 16, 16)]
                return 0

            lax.fori_loop(0, CH // 2, addrow, 0)
            pltpu.sync_copy(bufr2, out_hbm.at[wid, j])

    return edge_kernel(P, idx1)


def _relu(v):
    return jnp.maximum(v, 0.0)


def _padw(v, width):
    return jnp.concatenate(
        [v, jnp.zeros((v.shape[0], width - v.shape[1]), jnp.float32)], axis=1)


def _tc_stage_a(x, emb, Wn, bn, We, be):
    def body(x_ref, emb_ref, wn_ref, bn_ref, we_ref, be_ref, h_ref, e_ref):
        h_ref[...] = _relu(
            jnp.dot(x_ref[...], wn_ref[...],
                    preferred_element_type=jnp.float32) + bn_ref[...])
        e_ref[...] = _relu(
            jnp.dot(emb_ref[...], we_ref[...],
                    preferred_element_type=jnp.float32) + be_ref[...])

    return pl.pallas_call(
        body,
        out_shape=(jax.ShapeDtypeStruct((N, HID), jnp.float32),
                   jax.ShapeDtypeStruct((N, HID), jnp.float32)),
    )(x, emb, Wn, bn, We, be)


def _tc_stage_b(degp, h0, W0):
    def body(degp_ref, h0_ref, w_ref, dis_ref, g_ref):
        dsum = jnp.sum(degp_ref[...].reshape(NW, NP), axis=0)
        deg = jnp.reshape(dsum[0:N], (N, 1))
        dis = jnp.where(deg > 0.0, lax.rsqrt(jnp.maximum(deg, 1e-12)), 0.0)
        dis_ref[...] = dis
        hw = jnp.dot(h0_ref[...], w_ref[...],
                     preferred_element_type=jnp.float32)
        g_ref[...] = _padw(dis * hw, HG)

    return pl.pallas_call(
        body,
        out_shape=(jax.ShapeDtypeStruct((N, 1), jnp.float32),
                   jax.ShapeDtypeStruct((N, HG), jnp.float32)),
    )(degp, h0, W0)


def _arma_post(s_ref, dis, h_prev, V, b, gam, beta):
    m = dis * (s_ref[0] + s_ref[1])[0:N, 0:HID]
    h = _relu(m + jnp.dot(h_prev, V, preferred_element_type=jnp.float32) + b)
    mean = jnp.mean(h, axis=0)
    var = jnp.mean((h - mean) ** 2, axis=0)
    return gam * (h - mean) * lax.rsqrt(var + 1e-5) + beta


def _tc_stage_c(s0, dis, h0, V0, b0, g0p, beta0, W1conv):
    def body(s_ref, dis_ref, h0_ref, v_ref, b_ref, g_ref, bt_ref, w1_ref,
             h1_ref, g1_ref):
        h1 = _arma_post(s_ref, dis_ref[...], h0_ref[...], v_ref[...],
                        b_ref[...], g_ref[...], bt_ref[...])
        h1_ref[...] = h1
        hw = jnp.dot(h1, w1_ref[...], preferred_element_type=jnp.float32)
        g1_ref[...] = _padw(dis_ref[...] * hw, HG)

    return pl.pallas_call(
        body,
        out_shape=(jax.ShapeDtypeStruct((N, HID), jnp.float32),
                   jax.ShapeDtypeStruct((N, HG), jnp.float32)),
    )(s0, dis, h0, V0, b0, g0p, beta0, W1conv)


def _tc_stage_d(s1, dis, h1, V1, b1c, g1p, beta1, e0, W1):
    def body(s_ref, dis_ref, h1_ref, v_ref, b_ref, g_ref, bt_ref, e_ref,
             w1_ref, p_ref, q_ref):
        h2 = _arma_post(s_ref, dis_ref[...], h1_ref[...], v_ref[...],
                        b_ref[...], g_ref[...], bt_ref[...])
        w1 = w1_ref[...]
        A = w1[0:50] + w1[100:150] + w1[200:250]
        B = w1[50:100] + w1[150:200] + w1[250:300]
        C = w1[300:350] + w1[400:450] + w1[500:550]
        D = w1[350:400] + w1[450:500] + w1[550:600]
        e0v = e_ref[...]
        pv = (jnp.dot(h2, A, preferred_element_type=jnp.float32)
              + jnp.dot(e0v, B, preferred_element_type=jnp.float32))
        qv = (jnp.dot(h2, C, preferred_element_type=jnp.float32)
              + jnp.dot(e0v, D, preferred_element_type=jnp.float32))
        p_ref[...] = _padw(pv, HG)
        q_ref[...] = _padw(qv, HG)

    return pl.pallas_call(
        body,
        out_shape=(jax.ShapeDtypeStruct((N, HG), jnp.float32),
                   jax.ShapeDtypeStruct((N, HG), jnp.float32)),
    )(s1, dis, h1, V1, b1c, g1p, beta1, e0, W1)


def _tc_stage_e2(Rp, Rq, b1, W2, b2):
    BE2 = 4000

    def body(b1_ref, w2_ref, b2_ref, rp_ref, rq_ref, o_ref):
        rp = rp_ref[...]
        rq = rq_ref[...]
        w2 = w2_ref[...][:, 0]
        z0 = jnp.tanh(rp[:, 0:HID] + rq[:, 0:HID] + b1_ref[...])
        z1 = jnp.tanh(rp[:, HR:HR + HID] + rq[:, HR:HR + HID] + b1_ref[...])
        l0 = jnp.sum(z0 * w2, axis=1, keepdims=True) + b2_ref[0]
        l1 = jnp.sum(z1 * w2, axis=1, keepdims=True) + b2_ref[0]
        o_ref[...] = jax.nn.sigmoid(jnp.concatenate([l0, l1], axis=1))

    return pl.pallas_call(
        body,
        grid=(E // 2 // BE2,),
        in_specs=[
            pl.BlockSpec((HID,), lambda i: (0,)),
            pl.BlockSpec((HID, 1), lambda i: (0, 0)),
            pl.BlockSpec((1,), lambda i: (0,)),
            pl.BlockSpec((BE2, HG), lambda i: (i, 0)),
            pl.BlockSpec((BE2, HG), lambda i: (i, 0)),
        ],
        out_specs=pl.BlockSpec((BE2, 2), lambda i: (i, 0)),
        out_shape=jax.ShapeDtypeStruct((E // 2, 2), jnp.float32),
    )(b1, W2, b2, Rp, Rq)


def kernel(x, emb, edge_index, pedge_index, Wn, bn, We, be, W_conv0, V_conv0,
           b_conv0, g_bn0, beta_bn0, W_conv1, V_conv1, b_conv1, g_bn1,
           beta_bn1, W1, b1, W2, b2):
    src1 = edge_index[0]
    dst1 = edge_index[1]
    psrc1 = pedge_index[0]
    pdst1 = pedge_index[1]

    h0, e0 = _tc_stage_a(x, emb, Wn, bn, We, be)
    degp = _sc_degree(dst1)
    dis, g0 = _tc_stage_b(degp, h0, W_conv0)
    s0 = _sc_segsum(g0, src1, dst1).reshape(NC, NP, HR)
    h1, g1 = _tc_stage_c(s0, dis, h0, V_conv0, b_conv0, g_bn0, beta_bn0,
                         W_conv1)
    s1 = _sc_segsum(g1, src1, dst1).reshape(NC, NP, HR)
    P, Q = _tc_stage_d(s1, dis, h1, V_conv1, b_conv1, g_bn1, beta_bn1, e0, W1)
    Rp = _sc_edge1(P, psrc1).reshape(E // 2, HG)
    Rq = _sc_edge1(Q, pdst1).reshape(E // 2, HG)
    return _tc_stage_e2(Rp, Rq, b1, W2, b2).reshape(E)
